# baseline (speedup 1.0000x reference)
import functools

import jax
import jax.numpy as jnp
from jax import lax
from jax.experimental import pallas as pl
from jax.experimental.pallas import tpu as pltpu
from jax.experimental.pallas import tpu_sc as plsc

VOCAB = 1000000
D = 64
B = 16384
K = 20
NW = 32
BPW = B // NW
NC = 16
NCH = BPW // NC
NROWS = NC * K
NGD = 5
GR = NROWS // NGD

_LANES = tuple(range(16))


def _sc_body(u_hbm, v_hbm, pu_hbm, pv_hbm, nv_hbm, out_hbm,
             uidx, vidx, nidx, ub0, vb0, nb0, ub1, vb1, nb1,
             outp, outn, sem0, sem1):
    wid = lax.axis_index("s") * 2 + lax.axis_index("c")
    iota = lax.iota(jnp.int32, 16)

    pltpu.sync_copy(pu_hbm.at[wid], uidx)
    pltpu.sync_copy(pv_hbm.at[wid], vidx)
    pltpu.sync_copy(nv_hbm.at[wid], nidx)

    bufs = ((ub0, vb0, nb0, sem0), (ub1, vb1, nb1, sem1))

    def dmas(c, par):
        ub, vb, nb, sem = bufs[par]
        yield (u_hbm.at[uidx.at[c]], ub, sem)
        yield (v_hbm.at[vidx.at[c]], vb, sem)
        for j in range(NGD):
            yield (v_hbm.at[nidx.at[c * NGD + j]], nb.at[pl.ds(j * GR, GR)], sem)

    def fire(c, par):
        for s, d, m in dmas(c, par):
            pltpu.async_copy(s, d, m)

    def wait(c, par):
        for s, d, m in dmas(c, par):
            pltpu.make_async_copy(s, d, m).wait()

    nrow = tuple(iota * K + k for k in range(K))
    zero = jnp.zeros((16,), jnp.float32)

    def compute(c, par):
        ub, vb, nb, _ = bufs[par]

        @plsc.parallel_loop(0, D, unroll=4, carry=(zero,) * (K + 1))
        def accs(d, accs):
            dcol = jnp.full((16,), d, jnp.int32)
            uvec = plsc.load_gather(ub, [iota, dcol])
            pacc = accs[0] + uvec * plsc.load_gather(vb, [iota, dcol])
            naccs = tuple(
                accs[1 + k] + uvec * plsc.load_gather(nb, [nrow[k], dcol])
                for k in range(K))
            return (pacc,) + naccs
        outp[pl.ds(c * NC, NC)] = accs[0]
        lanevec = c * (NC * K) + iota * K
        for k in range(K):
            plsc.store_scatter(outn, [lanevec + k], -accs[1 + k])

    fire(0, 0)

    def pair_body(c2, _):
        c = c2 * 2
        fire(c + 1, 1)
        wait(c, 0)
        compute(c, 0)

        @pl.when(c + 2 < NCH)
        def _():
            fire(c + 2, 0)
        wait(c + 1, 1)
        compute(c + 1, 1)
        return 0

    lax.fori_loop(0, NCH // 2, pair_body, 0)

    pltpu.sync_copy(outp, out_hbm.at[pl.ds(wid * BPW, BPW)])
    pltpu.sync_copy(outn, out_hbm.at[pl.ds(B + wid * BPW * K, BPW * K)])


_sc_dots = functools.partial(
    pl.kernel,
    out_type=jax.ShapeDtypeStruct((B * (K + 1),), jnp.float32),
    mesh=plsc.VectorSubcoreMesh(core_axis_name="c", subcore_axis_name="s"),
    compiler_params=pltpu.CompilerParams(
        needs_layout_passes=False, use_tc_tiling_on_sc=False),
    scratch_types=[
        pltpu.VMEM((NCH, NC), jnp.int32),
        pltpu.VMEM((NCH, NC), jnp.int32),
        pltpu.VMEM((NCH * NGD, GR), jnp.int32),
        pltpu.VMEM((NC, D), jnp.float32),
        pltpu.VMEM((NC, D), jnp.float32),
        pltpu.VMEM((NROWS, D), jnp.float32),
        pltpu.VMEM((NC, D), jnp.float32),
        pltpu.VMEM((NC, D), jnp.float32),
        pltpu.VMEM((NROWS, D), jnp.float32),
        pltpu.VMEM((BPW,), jnp.float32),
        pltpu.VMEM((BPW * K,), jnp.float32),
        pltpu.SemaphoreType.DMA,
        pltpu.SemaphoreType.DMA,
    ],
)(_sc_body)


def _tc_body(x_ref, o_ref):
    o_ref[0, 0] = -jnp.sum(jax.nn.log_sigmoid(x_ref[...]))


_tc_reduce = pl.pallas_call(
    _tc_body,
    out_shape=jax.ShapeDtypeStruct((1, 1), jnp.float32),
    out_specs=pl.BlockSpec(memory_space=pltpu.SMEM),
)


def kernel(u_weight, v_weight, pos_u, pos_v, neg_v):
    pu = pos_u.astype(jnp.int32).reshape(NW, NCH, NC)
    pv = pos_v.astype(jnp.int32).reshape(NW, NCH, NC)
    nv = neg_v.astype(jnp.int32).reshape(NW, NCH * NGD, GR)
    dots = _sc_dots(u_weight, v_weight, pu, pv, nv)
    loss = _tc_reduce(dots.reshape(B * (K + 1) // 1024, 1024))
    return loss[0, 0]


# speedup vs baseline: 1.2483x; 1.2483x over previous
import functools

import jax
import jax.numpy as jnp
from jax import lax
from jax.experimental import pallas as pl
from jax.experimental.pallas import tpu as pltpu
from jax.experimental.pallas import tpu_sc as plsc

VOCAB = 1000000
D = 64
B = 16384
K = 20
NW = 32
BPW = B // NW
NC = 16
NCH = BPW // NC
NROWS = NC * K
NGD = 5
GR = NROWS // NGD
DP = D

_LANES = tuple(range(16))


def _sc_body(u_hbm, v_hbm, pu_hbm, pv_hbm, nv_hbm, out_hbm,
             uidx, vidx, nidx, ub0, vb0, nb0, ub1, vb1, nb1,
             outp, outn, sem0, sem1):
    wid = lax.axis_index("s") * 2 + lax.axis_index("c")
    iota = lax.iota(jnp.int32, 16)

    pltpu.sync_copy(pu_hbm.at[wid], uidx)
    pltpu.sync_copy(pv_hbm.at[wid], vidx)
    pltpu.sync_copy(nv_hbm.at[wid], nidx)

    bufs = ((ub0, vb0, nb0, sem0), (ub1, vb1, nb1, sem1))

    def dmas(c, par):
        ub, vb, nb, sem = bufs[par]
        yield (u_hbm.at[uidx.at[c]], ub, sem)
        yield (v_hbm.at[vidx.at[c]], vb, sem)
        for j in range(NGD):
            yield (v_hbm.at[nidx.at[c * NGD + j]], nb.at[pl.ds(j * GR, GR)], sem)

    def fire(c, par):
        for s, d, m in dmas(c, par):
            pltpu.async_copy(s, d, m)

    def wait(c, par):
        for s, d, m in dmas(c, par):
            pltpu.make_async_copy(s, d, m).wait()

    xs = tuple(jnp.bitwise_xor(iota, 1 << t) for t in range(4))
    m0 = iota == 0

    def redsum(acc):
        for x in xs:
            acc = acc + acc.at[x].get(mode="promise_in_bounds")
        return acc

    def compute(c, par):
        ub, vb, nb, _ = bufs[par]

        @plsc.parallel_loop(0, NC, unroll=2)
        def _(bb):
            u = [ub[bb, pl.ds(16 * j, 16)] for j in range(4)]
            v = [vb[bb, pl.ds(16 * j, 16)] for j in range(4)]
            r = redsum(u[0] * v[0] + u[1] * v[1] + u[2] * v[2] + u[3] * v[3])
            plsc.store_scatter(
                outp, [jnp.full((16,), c * NC + bb, jnp.int32)], r, mask=m0)
            for k in range(K):
                row = bb * K + k
                n = [nb[row, pl.ds(16 * j, 16)] for j in range(4)]
                r = redsum(u[0] * n[0] + u[1] * n[1] + u[2] * n[2] + u[3] * n[3])
                plsc.store_scatter(
                    outn, [jnp.full((16,), c * (NC * K) + row, jnp.int32)],
                    -r, mask=m0)

    fire(0, 0)

    def pair_body(c2, _):
        c = c2 * 2
        fire(c + 1, 1)
        wait(c, 0)
        compute(c, 0)

        @pl.when(c + 2 < NCH)
        def _():
            fire(c + 2, 0)
        wait(c + 1, 1)
        compute(c + 1, 1)
        return 0

    lax.fori_loop(0, NCH // 2, pair_body, 0)

    pltpu.sync_copy(outp, out_hbm.at[pl.ds(wid * BPW, BPW)])
    pltpu.sync_copy(outn, out_hbm.at[pl.ds(B + wid * BPW * K, BPW * K)])


_sc_dots = functools.partial(
    pl.kernel,
    out_type=jax.ShapeDtypeStruct((B * (K + 1),), jnp.float32),
    mesh=plsc.VectorSubcoreMesh(core_axis_name="c", subcore_axis_name="s"),
    compiler_params=pltpu.CompilerParams(
        needs_layout_passes=False, use_tc_tiling_on_sc=False),
    scratch_types=[
        pltpu.VMEM((NCH, NC), jnp.int32),
        pltpu.VMEM((NCH, NC), jnp.int32),
        pltpu.VMEM((NCH * NGD, GR), jnp.int32),
        pltpu.VMEM((NC, DP), jnp.float32),
        pltpu.VMEM((NC, DP), jnp.float32),
        pltpu.VMEM((NROWS, DP), jnp.float32),
        pltpu.VMEM((NC, DP), jnp.float32),
        pltpu.VMEM((NC, DP), jnp.float32),
        pltpu.VMEM((NROWS, DP), jnp.float32),
        pltpu.VMEM((BPW,), jnp.float32),
        pltpu.VMEM((BPW * K,), jnp.float32),
        pltpu.SemaphoreType.DMA,
        pltpu.SemaphoreType.DMA,
    ],
)(_sc_body)


def _tc_body(x_ref, o_ref):
    o_ref[0, 0] = -jnp.sum(jax.nn.log_sigmoid(x_ref[...]))


_tc_reduce = pl.pallas_call(
    _tc_body,
    out_shape=jax.ShapeDtypeStruct((1, 1), jnp.float32),
    out_specs=pl.BlockSpec(memory_space=pltpu.SMEM),
)


def kernel(u_weight, v_weight, pos_u, pos_v, neg_v):
    pu = pos_u.astype(jnp.int32).reshape(NW, NCH, NC)
    pv = pos_v.astype(jnp.int32).reshape(NW, NCH, NC)
    nv = neg_v.astype(jnp.int32).reshape(NW, NCH * NGD, GR)
    dots = _sc_dots(u_weight, v_weight, pu, pv, nv)
    loss = _tc_reduce(dots.reshape(B * (K + 1) // 1024, 1024))
    return loss[0, 0]
